# baseline (speedup 1.0000x reference)
import jax
import jax.numpy as jnp
from jax import lax
from jax.experimental import pallas as pl
from jax.experimental.pallas import tpu as pltpu
from jax.experimental.pallas import tpu_sc as plsc

N = 10000
D = 128
E = 320000
NC = 2
NS = 16
NW = NC * NS
CHUNK = 80
AR = 10240
RPT = AR // NS

_mesh = plsc.VectorSubcoreMesh(core_axis_name="c", subcore_axis_name="s")


def _fill(ref, nrows, val):
    v = jnp.full((16,), val, jnp.float32)

    def row(i, _):
        def col(j, _):
            ref[i, pl.ds(j * 16, 16)] = v
            return 0
        return lax.fori_loop(0, D // 16, col, 0)
    lax.fori_loop(0, nrows, row, 0)


def _make_agg(layer1):
    nchunk = (E // NS if layer1 else E // NW) // CHUNK

    def body(feat, srci, dsti, out, acc, *bufs):
        c = lax.axis_index("c")
        s = lax.axis_index("s")
        src_v = bufs[0:8]
        dst_v = bufs[8:16]
        rows = bufs[16:20]
        gsem = bufs[20:24]
        ssem = bufs[24:28]
        isem = bufs[28:36]
        base = (s * (E // NS)) if layer1 else ((c * NS + s) * (E // NW))

        _fill(rows[0], CHUNK, 0.0)
        for r in range(RPT // CHUNK):
            pltpu.sync_copy(rows[0], acc.at[pl.ds(s * RPT + r * CHUNK, CHUNK)])
        if layer1:
            for b in range(4):
                _fill(rows[b], CHUNK, 1.0)
        plsc.subcore_barrier()

        def load_idx(ci, i8):
            off = base + ci * CHUNK
            pltpu.async_copy(srci.at[pl.ds(off, CHUNK)], src_v[i8], isem[i8])
            pltpu.async_copy(dsti.at[pl.ds(off, CHUNK)], dst_v[i8], isem[i8])

        def wait_idx(ci, i8):
            off = base + ci * CHUNK
            pltpu.make_async_copy(srci.at[pl.ds(off, CHUNK)], src_v[i8],
                                  isem[i8]).wait()
            pltpu.make_async_copy(dsti.at[pl.ds(off, CHUNK)], dst_v[i8],
                                  isem[i8]).wait()

        def issue_gather(i8, b):
            def gath():
                pltpu.async_copy(feat.at[src_v[i8]], rows[b], gsem[b])
            if layer1:
                pl.when(c == 0)(gath)
            else:
                gath()

        def wait_gather(i8, b):
            def wait():
                pltpu.make_async_copy(feat.at[src_v[i8]], rows[b],
                                      gsem[b]).wait()
            if layer1:
                pl.when(c == 0)(wait)
            else:
                wait()

        def issue_scatter(i8, b):
            pltpu.async_copy(rows[b], acc.at[dst_v[i8]], ssem[b], add=True)

        def wait_scatter(i8, b):
            pltpu.make_async_copy(rows[b], acc.at[dst_v[i8]], ssem[b]).wait()

        for k in range(4):
            load_idx(k, k)
        for k in range(2):
            wait_idx(k, k)
            issue_gather(k, k)

        M = (nchunk // 8) * 8

        @pl.loop(0, M, step=8)
        def _(ci0):
            for b8 in range(8):
                ci = ci0 + b8
                b = b8 % 4

                @pl.when(ci >= 2)
                def _():
                    wait_scatter((b8 + 6) % 8, (b + 2) % 4)

                @pl.when(ci + 4 < nchunk)
                def _():
                    load_idx(ci + 4, (b8 + 4) % 8)

                @pl.when(ci + 2 < nchunk)
                def _():
                    wait_idx(ci + 2, (b8 + 2) % 8)
                    issue_gather((b8 + 2) % 8, (b + 2) % 4)

                wait_gather(b8, b)
                issue_scatter(b8, b)

        for k in range(M, nchunk):
            k8 = k % 8
            kb = k % 4
            if k >= 2:
                wait_scatter((k8 + 6) % 8, (kb + 2) % 4)
            if k + 4 < nchunk:
                load_idx(k + 4, (k8 + 4) % 8)
            if k + 2 < nchunk:
                wait_idx(k + 2, (k8 + 2) % 8)
                issue_gather((k8 + 2) % 8, (kb + 2) % 4)
            wait_gather(k8, kb)
            issue_scatter(k8, kb)
        for k in range(max(nchunk - 2, 0), nchunk):
            wait_scatter(k % 8, k % 4)

        plsc.subcore_barrier()
        pltpu.sync_copy(acc.at[pl.ds(s * RPT, RPT)],
                        out.at[c, pl.ds(s * RPT, RPT)])

    return pl.kernel(
        body,
        out_type=jax.ShapeDtypeStruct((NC, AR, D), jnp.float32),
        mesh=_mesh,
        scratch_types=(
            pltpu.VMEM_SHARED((AR, D), jnp.float32),
            *(pltpu.VMEM((CHUNK,), jnp.int32) for _ in range(16)),
            *(pltpu.VMEM((CHUNK, D), jnp.float32) for _ in range(4)),
            *(pltpu.SemaphoreType.DMA for _ in range(16)),
        ),
    )


_agg1 = _make_agg(True)
_agg2 = _make_agg(False)

_BLK = 1000


def _make_dense(two_partials, relu):
    def body(p_ref, c_ref, x_ref, wl_ref, wr_ref, b_ref, o_ref):
        if two_partials:
            psum = p_ref[0] + p_ref[1]
        else:
            psum = p_ref[0]
        cnt = jnp.maximum(c_ref[0, :, 0:1], 1.0)
        mean = psum / cnt
        acc = lax.dot_general(mean, wl_ref[...], (((1,), (1,)), ((), ())),
                              preferred_element_type=jnp.float32)
        acc = acc + lax.dot_general(x_ref[...], wr_ref[...],
                                    (((1,), (1,)), ((), ())),
                                    preferred_element_type=jnp.float32)
        acc = acc + b_ref[...]
        if relu:
            acc = jnp.maximum(acc, 0.0)
        o_ref[...] = acc

    np = NC if two_partials else 1
    return pl.pallas_call(
        body,
        grid=(N // _BLK,),
        in_specs=[
            pl.BlockSpec((np, _BLK, D), lambda i: (0, i, 0)),
            pl.BlockSpec((1, _BLK, D), lambda i: (1, i, 0)),
            pl.BlockSpec((_BLK, D), lambda i: (i, 0)),
            pl.BlockSpec((D, D), lambda i: (0, 0)),
            pl.BlockSpec((D, D), lambda i: (0, 0)),
            pl.BlockSpec((1, D), lambda i: (0, 0)),
        ],
        out_specs=pl.BlockSpec((_BLK, D), lambda i: (i, 0)),
        out_shape=jax.ShapeDtypeStruct((N, D), jnp.float32),
    )


_dense1 = _make_dense(False, True)
_dense2 = _make_dense(True, False)


@jax.jit
def kernel(x, edge_index, W1l, W1r, b1, W2l, W2r, b2):
    src = edge_index[0].astype(jnp.int32)
    dst = edge_index[1].astype(jnp.int32)
    p1 = _agg1(x, src, dst)
    h = _dense1(p1[0:1], p1, x, W1l, W1r, b1.reshape(1, D))
    p2 = _agg2(h, src, dst)
    return _dense2(p2, p1, h, W2l, W2r, b2.reshape(1, D))
